# baseline (device time: 34580 ns/iter reference)
import jax
import jax.numpy as jnp
from jax import lax
from jax.experimental import pallas as pl
from jax.experimental.pallas import tpu as pltpu

N_DEV = 8


def kernel(x, pi):
    _, m, n = x.shape
    C = 16
    mc = m // C

    def body(pi_ref, x_ref, out_ref, xv, qbuf, scbuf, qrecv, screcv, ov,
             load_sems, q_send_sems, q_recv_sems, sc_send_sems, sc_recv_sems,
             store_sems):
        my = lax.axis_index("i")
        dst = pi_ref[my]
        src = jnp.int32(0)
        for j in range(N_DEV):
            src = jnp.where(pi_ref[j] == my, jnp.int32(j), src)

        barrier = pltpu.get_barrier_semaphore()
        pl.semaphore_signal(
            barrier, inc=1, device_id=(src,),
            device_id_type=pl.DeviceIdType.MESH,
        )

        loads = []
        for c in range(C):
            ld = pltpu.make_async_copy(
                x_ref.at[0, pl.ds(c * mc, mc), :],
                xv.at[c % 2],
                load_sems.at[c % 2],
            )
            loads.append(ld)
        loads[0].start()

        rdmas = []
        for c in range(C):
            if c + 1 < C:
                loads[c + 1].start()
            loads[c].wait()
            ch = xv[c % 2]
            scale = jnp.maximum(jnp.max(jnp.abs(ch)), 1e-30)
            qbuf[c] = jnp.clip(
                jnp.round(ch * (127.0 / scale)), -127.0, 127.0
            ).astype(jnp.int8)
            scbuf[c] = jnp.full((8, 128), scale, jnp.float32)
            if c == 0:
                pl.semaphore_wait(barrier, 1)
            r_sc = pltpu.make_async_remote_copy(
                src_ref=scbuf.at[c],
                dst_ref=screcv.at[c],
                send_sem=sc_send_sems.at[c],
                recv_sem=sc_recv_sems.at[c],
                device_id=(dst,),
                device_id_type=pl.DeviceIdType.MESH,
            )
            r_q = pltpu.make_async_remote_copy(
                src_ref=qbuf.at[c],
                dst_ref=qrecv.at[c],
                send_sem=q_send_sems.at[c],
                recv_sem=q_recv_sems.at[c],
                device_id=(dst,),
                device_id_type=pl.DeviceIdType.MESH,
            )
            r_sc.start()
            r_q.start()
            rdmas.append((r_sc, r_q))

        stores = []
        for c in range(C):
            r_sc, r_q = rdmas[c]
            r_sc.wait_recv()
            r_q.wait_recv()
            if c >= 2:
                stores[c - 2].wait()
            scale_r = screcv[c, 0:1, 0:1] * (1.0 / 127.0)
            ov[c % 2] = (qrecv[c].astype(jnp.float32) * scale_r).astype(
                jnp.bfloat16
            )
            st = pltpu.make_async_copy(
                ov.at[c % 2],
                out_ref.at[0, pl.ds(c * mc, mc), :],
                store_sems.at[c % 2],
            )
            st.start()
            stores.append(st)

        for c in range(C - 2, C):
            stores[c].wait()
        for c in range(C):
            rdmas[c][0].wait_send()
            rdmas[c][1].wait_send()

    return pl.pallas_call(
        body,
        out_shape=jax.ShapeDtypeStruct(x.shape, jnp.bfloat16),
        in_specs=[
            pl.BlockSpec(memory_space=pltpu.SMEM),
            pl.BlockSpec(memory_space=pl.ANY),
        ],
        out_specs=pl.BlockSpec(memory_space=pl.ANY),
        scratch_shapes=[
            pltpu.VMEM((2, mc, n), jnp.float32),
            pltpu.VMEM((C, mc, n), jnp.int8),
            pltpu.VMEM((C, 8, 128), jnp.float32),
            pltpu.VMEM((C, mc, n), jnp.int8),
            pltpu.VMEM((C, 8, 128), jnp.float32),
            pltpu.VMEM((2, mc, n), jnp.bfloat16),
            pltpu.SemaphoreType.DMA((2,)),
            pltpu.SemaphoreType.DMA((C,)),
            pltpu.SemaphoreType.DMA((C,)),
            pltpu.SemaphoreType.DMA((C,)),
            pltpu.SemaphoreType.DMA((C,)),
            pltpu.SemaphoreType.DMA((2,)),
        ],
        compiler_params=pltpu.CompilerParams(collective_id=0),
    )(pi, x)


# device time: 34222 ns/iter; 1.0105x vs baseline; 1.0105x over previous
import jax
import jax.numpy as jnp
from jax import lax
from jax.experimental import pallas as pl
from jax.experimental.pallas import tpu as pltpu

N_DEV = 8


def kernel(x, pi):
    _, m, n = x.shape
    C = 8
    mc = m // C

    def body(pi_ref, x_ref, out_ref, xv, qbuf, scbuf, qrecv, screcv, ov,
             load_sems, q_send_sems, q_recv_sems, sc_send_sems, sc_recv_sems,
             store_sems):
        my = lax.axis_index("i")
        dst = pi_ref[my]
        src = jnp.int32(0)
        for j in range(N_DEV):
            src = jnp.where(pi_ref[j] == my, jnp.int32(j), src)

        barrier = pltpu.get_barrier_semaphore()
        pl.semaphore_signal(
            barrier, inc=1, device_id=(src,),
            device_id_type=pl.DeviceIdType.MESH,
        )

        loads = []
        for c in range(C):
            ld = pltpu.make_async_copy(
                x_ref.at[0, pl.ds(c * mc, mc), :],
                xv.at[c % 2],
                load_sems.at[c % 2],
            )
            loads.append(ld)
        loads[0].start()

        rdmas = []
        for c in range(C):
            if c + 1 < C:
                loads[c + 1].start()
            loads[c].wait()
            ch = xv[c % 2]
            scale = jnp.maximum(jnp.max(jnp.abs(ch)), 1e-30)
            qbuf[c] = jnp.clip(
                jnp.round(ch * (127.0 / scale)), -127.0, 127.0
            ).astype(jnp.int8)
            scbuf[c] = jnp.full((8, 128), scale, jnp.float32)
            if c == 0:
                pl.semaphore_wait(barrier, 1)
            r_sc = pltpu.make_async_remote_copy(
                src_ref=scbuf.at[c],
                dst_ref=screcv.at[c],
                send_sem=sc_send_sems.at[c],
                recv_sem=sc_recv_sems.at[c],
                device_id=(dst,),
                device_id_type=pl.DeviceIdType.MESH,
            )
            r_q = pltpu.make_async_remote_copy(
                src_ref=qbuf.at[c],
                dst_ref=qrecv.at[c],
                send_sem=q_send_sems.at[c],
                recv_sem=q_recv_sems.at[c],
                device_id=(dst,),
                device_id_type=pl.DeviceIdType.MESH,
            )
            r_sc.start()
            r_q.start()
            rdmas.append((r_sc, r_q))

        stores = []
        for c in range(C):
            r_sc, r_q = rdmas[c]
            r_sc.wait_recv()
            r_q.wait_recv()
            if c >= 2:
                stores[c - 2].wait()
            scale_r = screcv[c, 0:1, 0:1] * (1.0 / 127.0)
            ov[c % 2] = (qrecv[c].astype(jnp.float32) * scale_r).astype(
                jnp.bfloat16
            )
            st = pltpu.make_async_copy(
                ov.at[c % 2],
                out_ref.at[0, pl.ds(c * mc, mc), :],
                store_sems.at[c % 2],
            )
            st.start()
            stores.append(st)

        for c in range(C - 2, C):
            stores[c].wait()
        for c in range(C):
            rdmas[c][0].wait_send()
            rdmas[c][1].wait_send()

    return pl.pallas_call(
        body,
        out_shape=jax.ShapeDtypeStruct(x.shape, jnp.bfloat16),
        in_specs=[
            pl.BlockSpec(memory_space=pltpu.SMEM),
            pl.BlockSpec(memory_space=pl.ANY),
        ],
        out_specs=pl.BlockSpec(memory_space=pl.ANY),
        scratch_shapes=[
            pltpu.VMEM((2, mc, n), jnp.float32),
            pltpu.VMEM((C, mc, n), jnp.int8),
            pltpu.VMEM((C, 8, 128), jnp.float32),
            pltpu.VMEM((C, mc, n), jnp.int8),
            pltpu.VMEM((C, 8, 128), jnp.float32),
            pltpu.VMEM((2, mc, n), jnp.bfloat16),
            pltpu.SemaphoreType.DMA((2,)),
            pltpu.SemaphoreType.DMA((C,)),
            pltpu.SemaphoreType.DMA((C,)),
            pltpu.SemaphoreType.DMA((C,)),
            pltpu.SemaphoreType.DMA((C,)),
            pltpu.SemaphoreType.DMA((2,)),
        ],
        compiler_params=pltpu.CompilerParams(collective_id=0),
    )(pi, x)


# device time: 33904 ns/iter; 1.0199x vs baseline; 1.0094x over previous
import jax
import jax.numpy as jnp
from jax import lax
from jax.experimental import pallas as pl
from jax.experimental.pallas import tpu as pltpu

N_DEV = 8

SIZES = (64, 128, 256, 384, 448, 384, 256, 128)


def kernel(x, pi):
    _, m, n = x.shape
    assert sum(SIZES) == m
    C = len(SIZES)
    offs = [sum(SIZES[:c]) for c in range(C)]
    mx = max(SIZES)

    def body(pi_ref, x_ref, out_ref, xv, qbuf, scbuf, qrecv, screcv, ov,
             load_sems, q_send_sems, q_recv_sems, sc_send_sems, sc_recv_sems,
             store_sems):
        my = lax.axis_index("i")
        dst = pi_ref[my]
        src = jnp.int32(0)
        for j in range(N_DEV):
            src = jnp.where(pi_ref[j] == my, jnp.int32(j), src)

        barrier = pltpu.get_barrier_semaphore()
        pl.semaphore_signal(
            barrier, inc=1, device_id=(src,),
            device_id_type=pl.DeviceIdType.MESH,
        )

        loads = []
        for c in range(C):
            ld = pltpu.make_async_copy(
                x_ref.at[0, pl.ds(offs[c], SIZES[c]), :],
                xv.at[c % 2, pl.ds(0, SIZES[c]), :],
                load_sems.at[c % 2],
            )
            loads.append(ld)
        loads[0].start()

        rdmas = []
        for c in range(C):
            if c + 1 < C:
                loads[c + 1].start()
            loads[c].wait()
            ch = xv[c % 2, 0:SIZES[c], :]
            scale = jnp.maximum(jnp.max(jnp.abs(ch)), 1e-30)
            qbuf[pl.ds(offs[c], SIZES[c]), :] = jnp.clip(
                jnp.round(ch * (127.0 / scale)), -127.0, 127.0
            ).astype(jnp.int8)
            scbuf[c] = jnp.full((8, 128), scale, jnp.float32)
            if c == 0:
                pl.semaphore_wait(barrier, 1)
            r_sc = pltpu.make_async_remote_copy(
                src_ref=scbuf.at[c],
                dst_ref=screcv.at[c],
                send_sem=sc_send_sems.at[c],
                recv_sem=sc_recv_sems.at[c],
                device_id=(dst,),
                device_id_type=pl.DeviceIdType.MESH,
            )
            r_q = pltpu.make_async_remote_copy(
                src_ref=qbuf.at[pl.ds(offs[c], SIZES[c]), :],
                dst_ref=qrecv.at[pl.ds(offs[c], SIZES[c]), :],
                send_sem=q_send_sems.at[c],
                recv_sem=q_recv_sems.at[c],
                device_id=(dst,),
                device_id_type=pl.DeviceIdType.MESH,
            )
            r_sc.start()
            r_q.start()
            rdmas.append((r_sc, r_q))

        stores = []
        for c in range(C):
            r_sc, r_q = rdmas[c]
            r_sc.wait_recv()
            r_q.wait_recv()
            if c >= 2:
                stores[c - 2].wait()
            scale_r = screcv[c, 0:1, 0:1] * (1.0 / 127.0)
            ov[c % 2, 0:SIZES[c], :] = (
                qrecv[pl.ds(offs[c], SIZES[c]), :].astype(jnp.float32)
                * scale_r
            ).astype(jnp.bfloat16)
            st = pltpu.make_async_copy(
                ov.at[c % 2, pl.ds(0, SIZES[c]), :],
                out_ref.at[0, pl.ds(offs[c], SIZES[c]), :],
                store_sems.at[c % 2],
            )
            st.start()
            stores.append(st)

        for c in range(C - 2, C):
            stores[c].wait()
        for c in range(C):
            rdmas[c][0].wait_send()
            rdmas[c][1].wait_send()

    return pl.pallas_call(
        body,
        out_shape=jax.ShapeDtypeStruct(x.shape, jnp.bfloat16),
        in_specs=[
            pl.BlockSpec(memory_space=pltpu.SMEM),
            pl.BlockSpec(memory_space=pl.ANY),
        ],
        out_specs=pl.BlockSpec(memory_space=pl.ANY),
        scratch_shapes=[
            pltpu.VMEM((2, mx, n), jnp.float32),
            pltpu.VMEM((m, n), jnp.int8),
            pltpu.VMEM((C, 8, 128), jnp.float32),
            pltpu.VMEM((m, n), jnp.int8),
            pltpu.VMEM((C, 8, 128), jnp.float32),
            pltpu.VMEM((2, mx, n), jnp.bfloat16),
            pltpu.SemaphoreType.DMA((2,)),
            pltpu.SemaphoreType.DMA((C,)),
            pltpu.SemaphoreType.DMA((C,)),
            pltpu.SemaphoreType.DMA((C,)),
            pltpu.SemaphoreType.DMA((C,)),
            pltpu.SemaphoreType.DMA((2,)),
        ],
        compiler_params=pltpu.CompilerParams(collective_id=0),
    )(pi, x)
